# Initial kernel scaffold; baseline (speedup 1.0000x reference)
import jax
import jax.numpy as jnp
from jax.experimental import pallas as pl


def kernel(x, edge_index, W1, b1, W2, b2):
    raise NotImplementedError("write your pallas kernel here")


# speedup vs baseline: 7.3136x; 7.3136x over previous
import jax
import jax.numpy as jnp
from jax import lax
from jax.experimental import pallas as pl
from jax.experimental.pallas import tpu as pltpu
from jax.experimental.pallas import tpu_sc as plsc

N = 10000
E = 320000
DIM = 128
NPAD = 10240
EPAD = 327680
ROWS_PT = NPAD // 16
EDGE_CHUNK = 128
EPT = EPAD // 32


def _mesh():
    return plsc.VectorSubcoreMesh(core_axis_name="c", subcore_axis_name="s",
                                  num_cores=2, num_subcores=16)


def _fill_rows16(ref, nrows, width, value):
    v = jnp.full((16,), value, jnp.float32)
    for r in range(nrows):
        for q in range(width // 16):
            ref[r, pl.ds(q * 16, 16)] = v


def _deg_body(rowp_hbm, ones_hbm, h_hbm, hist, onesb, rowv, zb):
    c = lax.axis_index("c")
    s = lax.axis_index("s")
    _fill_rows16(zb, 16, DIM, 0.0)
    pltpu.sync_copy(ones_hbm, onesb)

    def zloop(k, _):
        pltpu.sync_copy(zb, hist.at[pl.ds(s * ROWS_PT + k * 16, 16)])
        return 0
    lax.fori_loop(0, ROWS_PT // 16, zloop, 0)
    plsc.subcore_barrier()

    base_e = (c * 16 + s) * EPT

    def chunk(j, _):
        pltpu.sync_copy(rowp_hbm.at[pl.ds(base_e + j * EDGE_CHUNK, EDGE_CHUNK)], rowv)
        pltpu.sync_copy(onesb, hist.at[rowv], add=True)
        return 0
    lax.fori_loop(0, EPT // EDGE_CHUNK, chunk, 0)
    plsc.subcore_barrier()

    pltpu.sync_copy(hist.at[pl.ds(s * ROWS_PT, ROWS_PT)],
                    h_hbm.at[c, pl.ds(s * ROWS_PT, ROWS_PT)])


def _deg_call():
  return pl.kernel(
    _deg_body,
    out_type=jax.ShapeDtypeStruct((2, NPAD, DIM), jnp.float32),
    mesh=_mesh(),
    scratch_types=[
        pltpu.VMEM_SHARED((NPAD, DIM), jnp.float32),
        pltpu.VMEM((EDGE_CHUNK, DIM), jnp.float32),
        pltpu.VMEM((EDGE_CHUNK,), jnp.int32),
        pltpu.VMEM((16, DIM), jnp.float32),
    ],
  )


def _agg_body(yp_hbm, rowp_hbm, colp_hbm, s_hbm,
              acc, rowbuf, colbuf, colv, rows, zb, sem):
    c = lax.axis_index("c")
    s = lax.axis_index("s")
    _fill_rows16(zb, 16, DIM, 0.0)

    def zloop(k, _):
        pltpu.sync_copy(zb, acc.at[pl.ds(s * ROWS_PT + k * 16, 16)])
        return 0
    lax.fori_loop(0, ROWS_PT // 16, zloop, 0)

    base_e = (c * 16 + s) * EPT
    pltpu.sync_copy(rowp_hbm.at[pl.ds(base_e, EPT)], rowbuf)
    pltpu.sync_copy(colp_hbm.at[pl.ds(base_e, EPT)], colbuf)
    plsc.subcore_barrier()

    def chunk(j, _):
        idxsl = rowbuf.at[pl.ds(j * EDGE_CHUNK, EDGE_CHUNK)]
        pltpu.async_copy(yp_hbm.at[idxsl], rows, sem).wait()

        def cp(q, _):
            colv[pl.ds(q * 16, 16)] = colbuf[pl.ds(j * EDGE_CHUNK + q * 16, 16)]
            return 0
        lax.fori_loop(0, EDGE_CHUNK // 16, cp, 0)
        pltpu.sync_copy(rows, acc.at[colv], add=True)
        return 0
    lax.fori_loop(0, EPT // EDGE_CHUNK, chunk, 0)
    plsc.subcore_barrier()

    pltpu.sync_copy(acc.at[pl.ds(s * ROWS_PT, ROWS_PT)],
                    s_hbm.at[c, pl.ds(s * ROWS_PT, ROWS_PT)])


def _agg_call():
  return pl.kernel(
    _agg_body,
    out_type=jax.ShapeDtypeStruct((2, NPAD, DIM), jnp.float32),
    mesh=_mesh(),
    scratch_types=[
        pltpu.VMEM_SHARED((NPAD, DIM), jnp.float32),
        pltpu.VMEM((EPT,), jnp.int32),
        pltpu.VMEM((EPT,), jnp.int32),
        pltpu.VMEM((EDGE_CHUNK,), jnp.int32),
        pltpu.VMEM((EDGE_CHUNK, DIM), jnp.float32),
        pltpu.VMEM((16, DIM), jnp.float32),
        pltpu.SemaphoreType.DMA,
    ],
  )


_BN = 1024


def _dis(h0_ref, h1_ref):
    deg = h0_ref[:, 0:1] + h1_ref[:, 0:1] + 1.0
    return lax.rsqrt(deg)


def _b1_body(x_ref, w1_ref, h0_ref, h1_ref, yp_ref):
    dis = _dis(h0_ref, h1_ref)
    y = lax.dot_general(x_ref[...], w1_ref[...], (((1,), (1,)), ((), ())),
                        preferred_element_type=jnp.float32)
    yp_ref[...] = dis * y


def _b1_call(xp, w1, h0, h1):
    return pl.pallas_call(
        _b1_body,
        grid=(NPAD // _BN,),
        in_specs=[
            pl.BlockSpec((_BN, DIM), lambda i: (i, 0)),
            pl.BlockSpec((DIM, DIM), lambda i: (0, 0)),
            pl.BlockSpec((_BN, DIM), lambda i: (i, 0)),
            pl.BlockSpec((_BN, DIM), lambda i: (i, 0)),
        ],
        out_specs=pl.BlockSpec((_BN, DIM), lambda i: (i, 0)),
        out_shape=jax.ShapeDtypeStruct((NPAD, DIM), jnp.float32),
    )(xp, w1, h0, h1)


def _b2_body(s0_ref, s1_ref, yp_ref, h0_ref, h1_ref, w2_ref, b1_ref, op_ref):
    dis = _dis(h0_ref, h1_ref)
    u = s0_ref[...] + s1_ref[...] + yp_ref[...]
    h = jnp.maximum(dis * u + b1_ref[...], 0.0)
    y2 = lax.dot_general(h, w2_ref[...], (((1,), (1,)), ((), ())),
                         preferred_element_type=jnp.float32)
    op_ref[...] = dis * y2


def _b2_call(s0, s1, yp, h0, h1, w2, b1):
    return pl.pallas_call(
        _b2_body,
        grid=(NPAD // _BN,),
        in_specs=[
            pl.BlockSpec((_BN, DIM), lambda i: (i, 0)),
            pl.BlockSpec((_BN, DIM), lambda i: (i, 0)),
            pl.BlockSpec((_BN, DIM), lambda i: (i, 0)),
            pl.BlockSpec((_BN, DIM), lambda i: (i, 0)),
            pl.BlockSpec((_BN, DIM), lambda i: (i, 0)),
            pl.BlockSpec((DIM, DIM), lambda i: (0, 0)),
            pl.BlockSpec((1, DIM), lambda i: (0, 0)),
        ],
        out_specs=pl.BlockSpec((_BN, DIM), lambda i: (i, 0)),
        out_shape=jax.ShapeDtypeStruct((NPAD, DIM), jnp.float32),
    )(s0, s1, yp, h0, h1, w2, b1)


def _b3_body(s0_ref, s1_ref, yp_ref, h0_ref, h1_ref, b2_ref, out_ref):
    dis = _dis(h0_ref, h1_ref)
    o = dis * (s0_ref[...] + s1_ref[...] + yp_ref[...]) + b2_ref[...]
    m = jnp.max(o, axis=1, keepdims=True)
    e = jnp.exp(o - m)
    z = jnp.sum(e, axis=1, keepdims=True)
    out_ref[...] = o - m - jnp.log(z)


def _b3_call(s0, s1, yp, h0, h1, b2):
    return pl.pallas_call(
        _b3_body,
        grid=(NPAD // _BN,),
        in_specs=[
            pl.BlockSpec((_BN, DIM), lambda i: (i, 0)),
            pl.BlockSpec((_BN, DIM), lambda i: (i, 0)),
            pl.BlockSpec((_BN, DIM), lambda i: (i, 0)),
            pl.BlockSpec((_BN, DIM), lambda i: (i, 0)),
            pl.BlockSpec((_BN, DIM), lambda i: (i, 0)),
            pl.BlockSpec((1, DIM), lambda i: (0, 0)),
        ],
        out_specs=pl.BlockSpec((_BN, DIM), lambda i: (i, 0)),
        out_shape=jax.ShapeDtypeStruct((NPAD, DIM), jnp.float32),
    )(s0, s1, yp, h0, h1, b2)


def kernel(x, edge_index, W1, b1, W2, b2):
    xp = jnp.concatenate([x, jnp.zeros((NPAD - N, DIM), jnp.float32)], axis=0)
    pad = jnp.full((EPAD - E,), N, jnp.int32)
    rowp = jnp.concatenate([edge_index[0], pad])
    colp = jnp.concatenate([edge_index[1], pad])

    hh = _deg_call()(rowp, jnp.ones((EDGE_CHUNK, DIM), jnp.float32))
    h0, h1 = hh[0], hh[1]
    yp = _b1_call(xp, W1, h0, h1)
    ss = _agg_call()(yp, rowp, colp)
    op = _b2_call(ss[0], ss[1], yp, h0, h1, W2, jnp.reshape(b1, (1, DIM)))
    tt = _agg_call()(op, rowp, colp)
    out = _b3_call(tt[0], tt[1], op, h0, h1, jnp.reshape(b2, (1, DIM)))
    return out[:N]


# speedup vs baseline: 8.0589x; 1.1019x over previous
import jax
import jax.numpy as jnp
from jax import lax
from jax.experimental import pallas as pl
from jax.experimental.pallas import tpu as pltpu
from jax.experimental.pallas import tpu_sc as plsc

N = 10000
E = 320000
DIM = 128
NPAD = 10240
EPAD = 327680
ROWS_PT = NPAD // 16
EDGE_CHUNK = 128
EPT = EPAD // 32
SUPER = 2560


def _mesh():
    return plsc.VectorSubcoreMesh(core_axis_name="c", subcore_axis_name="s",
                                  num_cores=2, num_subcores=16)


def _fill_rows16(ref, nrows, width, value):
    v = jnp.full((16,), value, jnp.float32)
    for r in range(nrows):
        for q in range(width // 16):
            ref[r, pl.ds(q * 16, 16)] = v


def _deg_body(rowp_hbm, ones_hbm, h_hbm, hist, onesb, rowv, zb):
    c = lax.axis_index("c")
    s = lax.axis_index("s")
    _fill_rows16(zb, 16, DIM, 0.0)
    pltpu.sync_copy(ones_hbm, onesb)

    def zloop(k, _):
        pltpu.sync_copy(zb, hist.at[pl.ds(s * ROWS_PT + k * 16, 16)])
        return 0
    lax.fori_loop(0, ROWS_PT // 16, zloop, 0)
    plsc.subcore_barrier()

    base_e = (c * 16 + s) * EPT

    def chunk(j, _):
        pltpu.sync_copy(rowp_hbm.at[pl.ds(base_e + j * EDGE_CHUNK, EDGE_CHUNK)], rowv)
        pltpu.sync_copy(onesb, hist.at[rowv], add=True)
        return 0
    lax.fori_loop(0, EPT // EDGE_CHUNK, chunk, 0)
    plsc.subcore_barrier()

    pltpu.sync_copy(hist.at[pl.ds(s * ROWS_PT, ROWS_PT)],
                    h_hbm.at[c, pl.ds(s * ROWS_PT, ROWS_PT)])


def _deg_call():
  return pl.kernel(
    _deg_body,
    out_type=jax.ShapeDtypeStruct((2, NPAD, DIM), jnp.float32),
    mesh=_mesh(),
    scratch_types=[
        pltpu.VMEM_SHARED((NPAD, DIM), jnp.float32),
        pltpu.VMEM((EDGE_CHUNK, DIM), jnp.float32),
        pltpu.VMEM((EDGE_CHUNK,), jnp.int32),
        pltpu.VMEM((16, DIM), jnp.float32),
    ],
  )


def _agg_body(yp_hbm, rowp_hbm, colp_hbm, s_hbm,
              acc, rowbuf, colbuf, colv, rows0, rows1, zb, sem0, sem1):
    c = lax.axis_index("c")
    s = lax.axis_index("s")
    _fill_rows16(zb, 16, DIM, 0.0)

    def zloop(k, _):
        pltpu.sync_copy(zb, acc.at[pl.ds(s * ROWS_PT + k * 16, 16)])
        return 0
    lax.fori_loop(0, ROWS_PT // 16, zloop, 0)

    base_e = (c * 16 + s) * EPT
    plsc.subcore_barrier()

    npairs = SUPER // EDGE_CHUNK // 2

    def fire(j, rbuf, sem):
        pltpu.async_copy(yp_hbm.at[rowbuf.at[pl.ds(j * EDGE_CHUNK, EDGE_CHUNK)]],
                         rbuf, sem)

    def wait(rbuf, sem):
        pltpu.make_async_copy(
            yp_hbm.at[rowbuf.at[pl.ds(0, EDGE_CHUNK)]], rbuf, sem).wait()

    def scat(j, rbuf):
        def cp(q, _):
            colv[pl.ds(q * 16, 16)] = colbuf[pl.ds(j * EDGE_CHUNK + q * 16, 16)]
            return 0
        lax.fori_loop(0, EDGE_CHUNK // 16, cp, 0)
        pltpu.sync_copy(rbuf, acc.at[colv], add=True)

    def superblock(t, _):
        sb = base_e + t * SUPER
        pltpu.sync_copy(rowp_hbm.at[pl.ds(sb, SUPER)], rowbuf)
        pltpu.sync_copy(colp_hbm.at[pl.ds(sb, SUPER)], colbuf)
        fire(0, rows0, sem0)

        def pair(p, _):
            j0 = p * 2
            fire(j0 + 1, rows1, sem1)
            wait(rows0, sem0)
            scat(j0, rows0)

            @pl.when(p < npairs - 1)
            def _():
                fire(j0 + 2, rows0, sem0)

            wait(rows1, sem1)
            scat(j0 + 1, rows1)
            return 0
        lax.fori_loop(0, npairs, pair, 0)
        return 0
    lax.fori_loop(0, EPT // SUPER, superblock, 0)
    plsc.subcore_barrier()

    pltpu.sync_copy(acc.at[pl.ds(s * ROWS_PT, ROWS_PT)],
                    s_hbm.at[c, pl.ds(s * ROWS_PT, ROWS_PT)])


def _agg_call():
  return pl.kernel(
    _agg_body,
    out_type=jax.ShapeDtypeStruct((2, NPAD, DIM), jnp.float32),
    mesh=_mesh(),
    scratch_types=[
        pltpu.VMEM_SHARED((NPAD, DIM), jnp.float32),
        pltpu.VMEM((SUPER,), jnp.int32),
        pltpu.VMEM((SUPER,), jnp.int32),
        pltpu.VMEM((EDGE_CHUNK,), jnp.int32),
        pltpu.VMEM((EDGE_CHUNK, DIM), jnp.float32),
        pltpu.VMEM((EDGE_CHUNK, DIM), jnp.float32),
        pltpu.VMEM((16, DIM), jnp.float32),
        pltpu.SemaphoreType.DMA,
        pltpu.SemaphoreType.DMA,
    ],
  )


_BN = 1024


def _dis(h0_ref, h1_ref):
    deg = h0_ref[:, 0:1] + h1_ref[:, 0:1] + 1.0
    return lax.rsqrt(deg)


def _b1_body(x_ref, w1_ref, h0_ref, h1_ref, yp_ref):
    dis = _dis(h0_ref, h1_ref)
    y = lax.dot_general(x_ref[...], w1_ref[...], (((1,), (1,)), ((), ())),
                        preferred_element_type=jnp.float32)
    yp_ref[...] = dis * y


def _b1_call(xp, w1, h0, h1):
    return pl.pallas_call(
        _b1_body,
        grid=(NPAD // _BN,),
        in_specs=[
            pl.BlockSpec((_BN, DIM), lambda i: (i, 0)),
            pl.BlockSpec((DIM, DIM), lambda i: (0, 0)),
            pl.BlockSpec((_BN, DIM), lambda i: (i, 0)),
            pl.BlockSpec((_BN, DIM), lambda i: (i, 0)),
        ],
        out_specs=pl.BlockSpec((_BN, DIM), lambda i: (i, 0)),
        out_shape=jax.ShapeDtypeStruct((NPAD, DIM), jnp.float32),
    )(xp, w1, h0, h1)


def _b2_body(s0_ref, s1_ref, yp_ref, h0_ref, h1_ref, w2_ref, b1_ref, op_ref):
    dis = _dis(h0_ref, h1_ref)
    u = s0_ref[...] + s1_ref[...] + yp_ref[...]
    h = jnp.maximum(dis * u + b1_ref[...], 0.0)
    y2 = lax.dot_general(h, w2_ref[...], (((1,), (1,)), ((), ())),
                         preferred_element_type=jnp.float32)
    op_ref[...] = dis * y2


def _b2_call(s0, s1, yp, h0, h1, w2, b1):
    return pl.pallas_call(
        _b2_body,
        grid=(NPAD // _BN,),
        in_specs=[
            pl.BlockSpec((_BN, DIM), lambda i: (i, 0)),
            pl.BlockSpec((_BN, DIM), lambda i: (i, 0)),
            pl.BlockSpec((_BN, DIM), lambda i: (i, 0)),
            pl.BlockSpec((_BN, DIM), lambda i: (i, 0)),
            pl.BlockSpec((_BN, DIM), lambda i: (i, 0)),
            pl.BlockSpec((DIM, DIM), lambda i: (0, 0)),
            pl.BlockSpec((1, DIM), lambda i: (0, 0)),
        ],
        out_specs=pl.BlockSpec((_BN, DIM), lambda i: (i, 0)),
        out_shape=jax.ShapeDtypeStruct((NPAD, DIM), jnp.float32),
    )(s0, s1, yp, h0, h1, w2, b1)


def _b3_body(s0_ref, s1_ref, yp_ref, h0_ref, h1_ref, b2_ref, out_ref):
    dis = _dis(h0_ref, h1_ref)
    o = dis * (s0_ref[...] + s1_ref[...] + yp_ref[...]) + b2_ref[...]
    m = jnp.max(o, axis=1, keepdims=True)
    e = jnp.exp(o - m)
    z = jnp.sum(e, axis=1, keepdims=True)
    out_ref[...] = o - m - jnp.log(z)


def _b3_call(s0, s1, yp, h0, h1, b2):
    return pl.pallas_call(
        _b3_body,
        grid=(NPAD // _BN,),
        in_specs=[
            pl.BlockSpec((_BN, DIM), lambda i: (i, 0)),
            pl.BlockSpec((_BN, DIM), lambda i: (i, 0)),
            pl.BlockSpec((_BN, DIM), lambda i: (i, 0)),
            pl.BlockSpec((_BN, DIM), lambda i: (i, 0)),
            pl.BlockSpec((_BN, DIM), lambda i: (i, 0)),
            pl.BlockSpec((1, DIM), lambda i: (0, 0)),
        ],
        out_specs=pl.BlockSpec((_BN, DIM), lambda i: (i, 0)),
        out_shape=jax.ShapeDtypeStruct((NPAD, DIM), jnp.float32),
    )(s0, s1, yp, h0, h1, b2)


def kernel(x, edge_index, W1, b1, W2, b2):
    xp = jnp.concatenate([x, jnp.zeros((NPAD - N, DIM), jnp.float32)], axis=0)
    pad = jnp.full((EPAD - E,), N, jnp.int32)
    rowp = jnp.concatenate([edge_index[0], pad])
    colp = jnp.concatenate([edge_index[1], pad])

    hh = _deg_call()(rowp, jnp.ones((EDGE_CHUNK, DIM), jnp.float32))
    h0, h1 = hh[0], hh[1]
    yp = _b1_call(xp, W1, h0, h1)
    ss = _agg_call()(yp, rowp, colp)
    op = _b2_call(ss[0], ss[1], yp, h0, h1, W2, jnp.reshape(b1, (1, DIM)))
    tt = _agg_call()(op, rowp, colp)
    out = _b3_call(tt[0], tt[1], op, h0, h1, jnp.reshape(b2, (1, DIM)))
    return out[:N]


# speedup vs baseline: 9.1407x; 1.1342x over previous
import jax
import jax.numpy as jnp
from jax import lax
from jax.experimental import pallas as pl
from jax.experimental.pallas import tpu as pltpu
from jax.experimental.pallas import tpu_sc as plsc

N = 10000
E = 320000
DIM = 128
NPAD = 10240
EPAD = 327680
ROWS_PT = NPAD // 16
EDGE_CHUNK = 128
EPT = EPAD // 32
SUPER = 2560


def _mesh():
    return plsc.VectorSubcoreMesh(core_axis_name="c", subcore_axis_name="s",
                                  num_cores=2, num_subcores=16)


def _fill_rows16(ref, nrows, width, value):
    v = jnp.full((16,), value, jnp.float32)
    for r in range(nrows):
        for q in range(width // 16):
            ref[r, pl.ds(q * 16, 16)] = v


def _deg_body(rowp_hbm, ones_hbm, zeros_hbm, h_hbm, hist, onesb, rowbuf,
              rowv0, rowv1, sem0, sem1):
    c = lax.axis_index("c")
    s = lax.axis_index("s")
    pltpu.sync_copy(ones_hbm, onesb)
    pltpu.sync_copy(zeros_hbm, hist.at[pl.ds(s * ROWS_PT, ROWS_PT)])
    base_e = (c * 16 + s) * EPT
    pltpu.sync_copy(rowp_hbm.at[pl.ds(base_e, EPT)], rowbuf)
    plsc.subcore_barrier()

    nchunks = EPT // EDGE_CHUNK

    def fill(j, rv):
        def cp(q, _):
            rv[pl.ds(q * 16, 16)] = rowbuf[pl.ds(j * EDGE_CHUNK + q * 16, 16)]
            return 0
        lax.fori_loop(0, EDGE_CHUNK // 16, cp, 0)

    def fire(rv, sem):
        pltpu.async_copy(onesb, hist.at[rv], sem, add=True)

    def wait(rv, sem):
        pltpu.make_async_copy(onesb, hist.at[rv], sem).wait()

    fill(0, rowv0)
    fire(rowv0, sem0)

    def pair(p, _):
        j0 = p * 2
        fill(j0 + 1, rowv1)
        fire(rowv1, sem1)
        wait(rowv0, sem0)

        @pl.when(p < nchunks // 2 - 1)
        def _():
            fill(j0 + 2, rowv0)
            fire(rowv0, sem0)

        wait(rowv1, sem1)
        return 0
    lax.fori_loop(0, nchunks // 2, pair, 0)
    plsc.subcore_barrier()

    pltpu.sync_copy(hist.at[pl.ds(s * ROWS_PT, ROWS_PT)],
                    h_hbm.at[c, pl.ds(s * ROWS_PT, ROWS_PT)])


def _deg_call():
  return pl.kernel(
    _deg_body,
    out_type=jax.ShapeDtypeStruct((2, NPAD, DIM), jnp.float32),
    mesh=_mesh(),
    scratch_types=[
        pltpu.VMEM_SHARED((NPAD, DIM), jnp.float32),
        pltpu.VMEM((EDGE_CHUNK, DIM), jnp.float32),
        pltpu.VMEM((EPT,), jnp.int32),
        pltpu.VMEM((EDGE_CHUNK,), jnp.int32),
        pltpu.VMEM((EDGE_CHUNK,), jnp.int32),
        pltpu.SemaphoreType.DMA,
        pltpu.SemaphoreType.DMA,
    ],
  )


def _agg_body(yp_hbm, rowp_hbm, colp_hbm, zeros_hbm, s_hbm,
              acc, rowbuf, colbuf, colv, rows0, rows1, sem0, sem1):
    c = lax.axis_index("c")
    s = lax.axis_index("s")
    pltpu.sync_copy(zeros_hbm, acc.at[pl.ds(s * ROWS_PT, ROWS_PT)])
    base_e = (c * 16 + s) * EPT
    plsc.subcore_barrier()

    npairs = SUPER // EDGE_CHUNK // 2

    def fire(j, rbuf, sem):
        pltpu.async_copy(yp_hbm.at[rowbuf.at[pl.ds(j * EDGE_CHUNK, EDGE_CHUNK)]],
                         rbuf, sem)

    def wait(rbuf, sem):
        pltpu.make_async_copy(
            yp_hbm.at[rowbuf.at[pl.ds(0, EDGE_CHUNK)]], rbuf, sem).wait()

    def scat(j, rbuf):
        def cp(q, _):
            colv[pl.ds(q * 16, 16)] = colbuf[pl.ds(j * EDGE_CHUNK + q * 16, 16)]
            return 0
        lax.fori_loop(0, EDGE_CHUNK // 16, cp, 0)
        pltpu.sync_copy(rbuf, acc.at[colv], add=True)

    def superblock(t, _):
        sb = base_e + t * SUPER
        pltpu.sync_copy(rowp_hbm.at[pl.ds(sb, SUPER)], rowbuf)
        pltpu.sync_copy(colp_hbm.at[pl.ds(sb, SUPER)], colbuf)
        fire(0, rows0, sem0)

        def pair(p, _):
            j0 = p * 2
            fire(j0 + 1, rows1, sem1)
            wait(rows0, sem0)
            scat(j0, rows0)

            @pl.when(p < npairs - 1)
            def _():
                fire(j0 + 2, rows0, sem0)

            wait(rows1, sem1)
            scat(j0 + 1, rows1)
            return 0
        lax.fori_loop(0, npairs, pair, 0)
        return 0
    lax.fori_loop(0, EPT // SUPER, superblock, 0)
    plsc.subcore_barrier()

    pltpu.sync_copy(acc.at[pl.ds(s * ROWS_PT, ROWS_PT)],
                    s_hbm.at[c, pl.ds(s * ROWS_PT, ROWS_PT)])


def _agg_call():
  return pl.kernel(
    _agg_body,
    out_type=jax.ShapeDtypeStruct((2, NPAD, DIM), jnp.float32),
    mesh=_mesh(),
    scratch_types=[
        pltpu.VMEM_SHARED((NPAD, DIM), jnp.float32),
        pltpu.VMEM((SUPER,), jnp.int32),
        pltpu.VMEM((SUPER,), jnp.int32),
        pltpu.VMEM((EDGE_CHUNK,), jnp.int32),
        pltpu.VMEM((EDGE_CHUNK, DIM), jnp.float32),
        pltpu.VMEM((EDGE_CHUNK, DIM), jnp.float32),
        pltpu.SemaphoreType.DMA,
        pltpu.SemaphoreType.DMA,
    ],
  )


_BN = 1024


def _dis(h0_ref, h1_ref):
    deg = h0_ref[:, 0:1] + h1_ref[:, 0:1] + 1.0
    return lax.rsqrt(deg)


def _b1_body(x_ref, w1_ref, h0_ref, h1_ref, yp_ref):
    dis = _dis(h0_ref, h1_ref)
    y = lax.dot_general(x_ref[...], w1_ref[...], (((1,), (1,)), ((), ())),
                        preferred_element_type=jnp.float32)
    yp_ref[...] = dis * y


def _b1_call(xp, w1, h0, h1):
    return pl.pallas_call(
        _b1_body,
        grid=(NPAD // _BN,),
        in_specs=[
            pl.BlockSpec((_BN, DIM), lambda i: (i, 0)),
            pl.BlockSpec((DIM, DIM), lambda i: (0, 0)),
            pl.BlockSpec((_BN, DIM), lambda i: (i, 0)),
            pl.BlockSpec((_BN, DIM), lambda i: (i, 0)),
        ],
        out_specs=pl.BlockSpec((_BN, DIM), lambda i: (i, 0)),
        out_shape=jax.ShapeDtypeStruct((NPAD, DIM), jnp.float32),
    )(xp, w1, h0, h1)


def _b2_body(s0_ref, s1_ref, yp_ref, h0_ref, h1_ref, w2_ref, b1_ref, op_ref):
    dis = _dis(h0_ref, h1_ref)
    u = s0_ref[...] + s1_ref[...] + yp_ref[...]
    h = jnp.maximum(dis * u + b1_ref[...], 0.0)
    y2 = lax.dot_general(h, w2_ref[...], (((1,), (1,)), ((), ())),
                         preferred_element_type=jnp.float32)
    op_ref[...] = dis * y2


def _b2_call(s0, s1, yp, h0, h1, w2, b1):
    return pl.pallas_call(
        _b2_body,
        grid=(NPAD // _BN,),
        in_specs=[
            pl.BlockSpec((_BN, DIM), lambda i: (i, 0)),
            pl.BlockSpec((_BN, DIM), lambda i: (i, 0)),
            pl.BlockSpec((_BN, DIM), lambda i: (i, 0)),
            pl.BlockSpec((_BN, DIM), lambda i: (i, 0)),
            pl.BlockSpec((_BN, DIM), lambda i: (i, 0)),
            pl.BlockSpec((DIM, DIM), lambda i: (0, 0)),
            pl.BlockSpec((1, DIM), lambda i: (0, 0)),
        ],
        out_specs=pl.BlockSpec((_BN, DIM), lambda i: (i, 0)),
        out_shape=jax.ShapeDtypeStruct((NPAD, DIM), jnp.float32),
    )(s0, s1, yp, h0, h1, w2, b1)


def _b3_body(s0_ref, s1_ref, yp_ref, h0_ref, h1_ref, b2_ref, out_ref):
    dis = _dis(h0_ref, h1_ref)
    o = dis * (s0_ref[...] + s1_ref[...] + yp_ref[...]) + b2_ref[...]
    m = jnp.max(o, axis=1, keepdims=True)
    e = jnp.exp(o - m)
    z = jnp.sum(e, axis=1, keepdims=True)
    out_ref[...] = o - m - jnp.log(z)


def _b3_call(s0, s1, yp, h0, h1, b2):
    return pl.pallas_call(
        _b3_body,
        grid=(NPAD // _BN,),
        in_specs=[
            pl.BlockSpec((_BN, DIM), lambda i: (i, 0)),
            pl.BlockSpec((_BN, DIM), lambda i: (i, 0)),
            pl.BlockSpec((_BN, DIM), lambda i: (i, 0)),
            pl.BlockSpec((_BN, DIM), lambda i: (i, 0)),
            pl.BlockSpec((_BN, DIM), lambda i: (i, 0)),
            pl.BlockSpec((1, DIM), lambda i: (0, 0)),
        ],
        out_specs=pl.BlockSpec((_BN, DIM), lambda i: (i, 0)),
        out_shape=jax.ShapeDtypeStruct((NPAD, DIM), jnp.float32),
    )(s0, s1, yp, h0, h1, b2)


def kernel(x, edge_index, W1, b1, W2, b2):
    xp = jnp.concatenate([x, jnp.zeros((NPAD - N, DIM), jnp.float32)], axis=0)
    pad = jnp.full((EPAD - E,), N, jnp.int32)
    rowp = jnp.concatenate([edge_index[0], pad])
    colp = jnp.concatenate([edge_index[1], pad])

    ones = jnp.ones((EDGE_CHUNK, DIM), jnp.float32)
    zeros = jnp.zeros((ROWS_PT, DIM), jnp.float32)
    hh = _deg_call()(rowp, ones, zeros)
    h0, h1 = hh[0], hh[1]
    yp = _b1_call(xp, W1, h0, h1)
    ss = _agg_call()(yp, rowp, colp, zeros)
    op = _b2_call(ss[0], ss[1], yp, h0, h1, W2, jnp.reshape(b1, (1, DIM)))
    tt = _agg_call()(op, rowp, colp, zeros)
    out = _b3_call(tt[0], tt[1], op, h0, h1, jnp.reshape(b2, (1, DIM)))
    return out[:N]


# speedup vs baseline: 18.2812x; 2.0000x over previous
import jax
import jax.numpy as jnp
from jax import lax
from jax.experimental import pallas as pl
from jax.experimental.pallas import tpu as pltpu
from jax.experimental.pallas import tpu_sc as plsc

N = 10000
E = 320000
DIM = 128
HALF = 64
NPAD = 10240
EPAD = 327680
ROWS_PT = NPAD // 16
EDGE_CHUNK = 128
EPT = EPAD // 32
SUPER = 2560


def _mesh():
    return plsc.VectorSubcoreMesh(core_axis_name="c", subcore_axis_name="s",
                                  num_cores=2, num_subcores=16)


def _fill_rows16(ref, nrows, width, value):
    v = jnp.full((16,), value, jnp.float32)
    for r in range(nrows):
        for q in range(width // 16):
            ref[r, pl.ds(q * 16, 16)] = v


def _deg_body(rowp_hbm, ones_hbm, zeros_hbm, h_hbm, hist, onesb, rowbuf,
              rowv0, rowv1, sem0, sem1):
    c = lax.axis_index("c")
    s = lax.axis_index("s")
    pltpu.sync_copy(ones_hbm, onesb)
    pltpu.sync_copy(zeros_hbm, hist.at[pl.ds(s * ROWS_PT, ROWS_PT)])
    base_e = (c * 16 + s) * EPT
    pltpu.sync_copy(rowp_hbm.at[pl.ds(base_e, EPT)], rowbuf)
    plsc.subcore_barrier()

    nchunks = EPT // EDGE_CHUNK

    def fill(j, rv):
        def cp(q, _):
            rv[pl.ds(q * 16, 16)] = rowbuf[pl.ds(j * EDGE_CHUNK + q * 16, 16)]
            return 0
        lax.fori_loop(0, EDGE_CHUNK // 16, cp, 0)

    def fire(rv, sem):
        pltpu.async_copy(onesb, hist.at[rv], sem, add=True)

    def wait(rv, sem):
        pltpu.make_async_copy(onesb, hist.at[rv], sem).wait()

    fill(0, rowv0)
    fire(rowv0, sem0)

    def pair(p, _):
        j0 = p * 2
        fill(j0 + 1, rowv1)
        fire(rowv1, sem1)
        wait(rowv0, sem0)

        @pl.when(p < nchunks // 2 - 1)
        def _():
            fill(j0 + 2, rowv0)
            fire(rowv0, sem0)

        wait(rowv1, sem1)
        return 0
    lax.fori_loop(0, nchunks // 2, pair, 0)
    plsc.subcore_barrier()

    pltpu.sync_copy(hist.at[pl.ds(s * ROWS_PT, ROWS_PT)],
                    h_hbm.at[c, pl.ds(s * ROWS_PT, ROWS_PT)])


def _deg_call():
  return pl.kernel(
    _deg_body,
    out_type=jax.ShapeDtypeStruct((2, NPAD, DIM), jnp.float32),
    mesh=_mesh(),
    scratch_types=[
        pltpu.VMEM_SHARED((NPAD, DIM), jnp.float32),
        pltpu.VMEM((EDGE_CHUNK, DIM), jnp.float32),
        pltpu.VMEM((EPT,), jnp.int32),
        pltpu.VMEM((EDGE_CHUNK,), jnp.int32),
        pltpu.VMEM((EDGE_CHUNK,), jnp.int32),
        pltpu.SemaphoreType.DMA,
        pltpu.SemaphoreType.DMA,
    ],
  )


def _agg_body(yps_hbm, rowp_hbm, colp_hbm, zeros_hbm, s_hbm,
              ytab, acc, rowbuf, colbuf, colv, rows0, rows1, sem0, sem1):
    c = lax.axis_index("c")
    s = lax.axis_index("s")
    pltpu.sync_copy(zeros_hbm, acc.at[pl.ds(s * ROWS_PT, ROWS_PT)])
    pltpu.sync_copy(yps_hbm.at[c, pl.ds(s * ROWS_PT, ROWS_PT)],
                    ytab.at[pl.ds(s * ROWS_PT, ROWS_PT)])
    base_e = s * (EPAD // 16)
    plsc.subcore_barrier()

    npairs = SUPER // EDGE_CHUNK // 2

    def fire(j, rbuf, sem):
        pltpu.async_copy(ytab.at[rowbuf.at[pl.ds(j * EDGE_CHUNK, EDGE_CHUNK)]],
                         rbuf, sem)

    def wait(rbuf, sem):
        pltpu.make_async_copy(
            ytab.at[rowbuf.at[pl.ds(0, EDGE_CHUNK)]], rbuf, sem).wait()

    def scat(j, rbuf):
        def cp(q, _):
            colv[pl.ds(q * 16, 16)] = colbuf[pl.ds(j * EDGE_CHUNK + q * 16, 16)]
            return 0
        lax.fori_loop(0, EDGE_CHUNK // 16, cp, 0)
        pltpu.sync_copy(rbuf, acc.at[colv], add=True)

    def superblock(t, _):
        sb = base_e + t * SUPER
        pltpu.sync_copy(rowp_hbm.at[pl.ds(sb, SUPER)], rowbuf)
        pltpu.sync_copy(colp_hbm.at[pl.ds(sb, SUPER)], colbuf)
        fire(0, rows0, sem0)

        def pair(p, _):
            j0 = p * 2
            fire(j0 + 1, rows1, sem1)
            wait(rows0, sem0)
            scat(j0, rows0)

            @pl.when(p < npairs - 1)
            def _():
                fire(j0 + 2, rows0, sem0)

            wait(rows1, sem1)
            scat(j0 + 1, rows1)
            return 0
        lax.fori_loop(0, npairs, pair, 0)
        return 0
    lax.fori_loop(0, (EPAD // 16) // SUPER, superblock, 0)
    plsc.subcore_barrier()

    pltpu.sync_copy(acc.at[pl.ds(s * ROWS_PT, ROWS_PT)],
                    s_hbm.at[c, pl.ds(s * ROWS_PT, ROWS_PT)])


def _agg_call():
  return pl.kernel(
    _agg_body,
    out_type=jax.ShapeDtypeStruct((2, NPAD, HALF), jnp.float32),
    mesh=_mesh(),
    compiler_params=pltpu.CompilerParams(use_tc_tiling_on_sc=False),
    scratch_types=[
        pltpu.VMEM_SHARED((NPAD, HALF), jnp.float32),
        pltpu.VMEM_SHARED((NPAD, HALF), jnp.float32),
        pltpu.VMEM((SUPER,), jnp.int32),
        pltpu.VMEM((SUPER,), jnp.int32),
        pltpu.VMEM((EDGE_CHUNK,), jnp.int32),
        pltpu.VMEM((EDGE_CHUNK, HALF), jnp.float32),
        pltpu.VMEM((EDGE_CHUNK, HALF), jnp.float32),
        pltpu.SemaphoreType.DMA,
        pltpu.SemaphoreType.DMA,
    ],
  )


_BN = 1024


def _dis(h0_ref, h1_ref):
    deg = h0_ref[:, 0:1] + h1_ref[:, 0:1] + 1.0
    return lax.rsqrt(deg)


_SPLIT_SPEC = pl.BlockSpec((2, _BN, HALF), lambda i: (0, i, 0))


def _b1_body(x_ref, w1_ref, h0_ref, h1_ref, yp_ref):
    dis = _dis(h0_ref, h1_ref)
    y = lax.dot_general(x_ref[...], w1_ref[...], (((1,), (1,)), ((), ())),
                        preferred_element_type=jnp.float32)
    yp = dis * y
    yp_ref[0, ...] = yp[:, :HALF]
    yp_ref[1, ...] = yp[:, HALF:]


def _b1_call(xp, w1, h0, h1):
    return pl.pallas_call(
        _b1_body,
        grid=(NPAD // _BN,),
        in_specs=[
            pl.BlockSpec((_BN, DIM), lambda i: (i, 0)),
            pl.BlockSpec((DIM, DIM), lambda i: (0, 0)),
            pl.BlockSpec((_BN, DIM), lambda i: (i, 0)),
            pl.BlockSpec((_BN, DIM), lambda i: (i, 0)),
        ],
        out_specs=_SPLIT_SPEC,
        out_shape=jax.ShapeDtypeStruct((2, NPAD, HALF), jnp.float32),
    )(xp, w1, h0, h1)


def _b2_body(s_ref, yp_ref, h0_ref, h1_ref, w2_ref, b1_ref, op_ref):
    dis = _dis(h0_ref, h1_ref)
    u = jnp.concatenate([s_ref[0] + yp_ref[0], s_ref[1] + yp_ref[1]], axis=1)
    h = jnp.maximum(dis * u + b1_ref[...], 0.0)
    y2 = lax.dot_general(h, w2_ref[...], (((1,), (1,)), ((), ())),
                         preferred_element_type=jnp.float32)
    yp2 = dis * y2
    op_ref[0, ...] = yp2[:, :HALF]
    op_ref[1, ...] = yp2[:, HALF:]


def _b2_call(ss, yps, h0, h1, w2, b1):
    return pl.pallas_call(
        _b2_body,
        grid=(NPAD // _BN,),
        in_specs=[
            _SPLIT_SPEC,
            _SPLIT_SPEC,
            pl.BlockSpec((_BN, DIM), lambda i: (i, 0)),
            pl.BlockSpec((_BN, DIM), lambda i: (i, 0)),
            pl.BlockSpec((DIM, DIM), lambda i: (0, 0)),
            pl.BlockSpec((1, DIM), lambda i: (0, 0)),
        ],
        out_specs=_SPLIT_SPEC,
        out_shape=jax.ShapeDtypeStruct((2, NPAD, HALF), jnp.float32),
    )(ss, yps, h0, h1, w2, b1)


def _b3_body(s_ref, yp_ref, h0_ref, h1_ref, b2_ref, out_ref):
    dis = _dis(h0_ref, h1_ref)
    u = jnp.concatenate([s_ref[0] + yp_ref[0], s_ref[1] + yp_ref[1]], axis=1)
    o = dis * u + b2_ref[...]
    m = jnp.max(o, axis=1, keepdims=True)
    e = jnp.exp(o - m)
    z = jnp.sum(e, axis=1, keepdims=True)
    out_ref[...] = o - m - jnp.log(z)


def _b3_call(tt, yps, h0, h1, b2):
    return pl.pallas_call(
        _b3_body,
        grid=(NPAD // _BN,),
        in_specs=[
            _SPLIT_SPEC,
            _SPLIT_SPEC,
            pl.BlockSpec((_BN, DIM), lambda i: (i, 0)),
            pl.BlockSpec((_BN, DIM), lambda i: (i, 0)),
            pl.BlockSpec((1, DIM), lambda i: (0, 0)),
        ],
        out_specs=pl.BlockSpec((_BN, DIM), lambda i: (i, 0)),
        out_shape=jax.ShapeDtypeStruct((NPAD, DIM), jnp.float32),
    )(tt, yps, h0, h1, b2)


def kernel(x, edge_index, W1, b1, W2, b2):
    xp = jnp.concatenate([x, jnp.zeros((NPAD - N, DIM), jnp.float32)], axis=0)
    pad = jnp.full((EPAD - E,), N, jnp.int32)
    rowp = jnp.concatenate([edge_index[0], pad])
    colp = jnp.concatenate([edge_index[1], pad])

    ones = jnp.ones((EDGE_CHUNK, DIM), jnp.float32)
    zeros_d = jnp.zeros((ROWS_PT, DIM), jnp.float32)
    zeros_h = jnp.zeros((ROWS_PT, HALF), jnp.float32)
    hh = _deg_call()(rowp, ones, zeros_d)
    h0, h1 = hh[0], hh[1]
    yps = _b1_call(xp, W1, h0, h1)
    ss = _agg_call()(yps, rowp, colp, zeros_h)
    ops = _b2_call(ss, yps, h0, h1, W2, jnp.reshape(b1, (1, DIM)))
    tt = _agg_call()(ops, rowp, colp, zeros_h)
    out = _b3_call(tt, ops, h0, h1, jnp.reshape(b2, (1, DIM)))
    return out[:N]
